# baseline (speedup 1.0000x reference)
import functools

import jax
import jax.numpy as jnp
from jax import lax
from jax.experimental import pallas as pl
from jax.experimental.pallas import tpu as pltpu
from jax.experimental.pallas import tpu_sc as plsc

_B = 16384
_D = 32
_H = 64
_L = 16
_V = 1000000
_MAIN = 999936
_RANGE = 31360
_WIN = 768
_NWIN = 41
_WCAP = 256
_WPAD = 288
_SB = 32
_NCHUNK = 8
_CHK = 2048


def _sc_gather(table3, tail3, idx):
    mesh = plsc.VectorSubcoreMesh(core_axis_name="c", subcore_axis_name="s")

    @functools.partial(
        pl.kernel,
        mesh=mesh,
        out_type=jax.ShapeDtypeStruct((_B, 128), jnp.float32),
        scratch_types=[
            pltpu.VMEM((_B,), jnp.int32),
            pltpu.VMEM((_B + 32,), jnp.int32),
            pltpu.VMEM(((_NWIN + 1) * _WPAD,), jnp.int32),
            pltpu.VMEM((2, 4, 8, _WIN), jnp.float32),
            pltpu.VMEM((2, _SB, 128), jnp.float32),
            pltpu.VMEM((2, _SB), jnp.int32),
            pltpu.SMEM((_NWIN + 1,), jnp.int32),
            pltpu.SemaphoreType.DMA,
            pltpu.SemaphoreType.DMA,
            pltpu.SemaphoreType.DMA,
        ],
        compiler_params=pltpu.CompilerParams(needs_layout_passes=False),
    )
    def gather_kernel(table_hbm, tail_hbm, idx_hbm, out_hbm, idx_v,
                      l1_v, wlist_v, buf_v, rows_v, pos_v, counts_s,
                      sem0, sem1, semw):
        wid = lax.axis_index("s") * 2 + lax.axis_index("c")
        lo = wid * _RANGE
        relmain = _MAIN - lo
        lane = lax.iota(jnp.int32, _L)
        neg1 = jnp.full((_L,), -1, jnp.int32)
        sems = (sem0, sem1)

        def win_start(k):
            return jnp.minimum(lo + k * _WIN, _MAIN - _WIN)

        def issue(k, b):
            pltpu.async_copy(
                table_hbm.at[:, :, pl.ds(win_start(k), _WIN)],
                buf_v.at[b], sems[b])

        def drain(k, b):
            pltpu.make_async_copy(
                table_hbm.at[:, :, pl.ds(win_start(k), _WIN)],
                buf_v.at[b], sems[b]).wait()

        issue(0, 0)
        issue(1, 1)

        pltpu.sync_copy(idx_hbm, idx_v)

        def l1_body(g, n):
            ids = idx_v[pl.ds(g * _L, _L)]
            rel = ids - lo
            mask = (rel >= 0) & (rel < _RANGE)
            pos = lane + g * _L
            packed = lax.shift_left(rel, 14) | pos
            plsc.store_compressed(l1_v.at[pl.ds(n, _L)], packed, mask=mask)
            return n + plsc.all_reduce_population_count(mask)[0]

        n1 = lax.fori_loop(0, _B // _L, l1_body, jnp.int32(0))

        def l2_body(g, counts):
            packed1 = l1_v[pl.ds(g * _L, _L)]
            rel = lax.shift_right_logical(packed1, 14)
            pos = lax.bitwise_and(packed1, 0x3FFF)
            valid = (lane + g * _L) < n1
            vmain = valid & (rel < relmain)
            out_counts = []
            for k in range(_NWIN):
                m = vmain & (rel >= k * _WIN) & (rel < (k + 1) * _WIN)
                packed = lax.shift_left(rel - k * _WIN, 16) | pos
                nk = counts[k]
                plsc.store_compressed(
                    wlist_v.at[pl.ds(k * _WPAD + nk, _L)], packed, mask=m)
                out_counts.append(
                    jnp.minimum(nk + plsc.all_reduce_population_count(m)[0],
                                _WCAP))
            m = valid & (rel >= relmain) & (rel < _RANGE)
            packed = lax.shift_left(rel - relmain, 16) | pos
            nk = counts[_NWIN]
            plsc.store_compressed(
                wlist_v.at[pl.ds(_NWIN * _WPAD + nk, _L)], packed, mask=m)
            out_counts.append(
                jnp.minimum(nk + plsc.all_reduce_population_count(m)[0],
                            _WCAP))
            return tuple(out_counts)

        counts = lax.fori_loop(
            0, lax.div(n1 + (_L - 1), _L), l2_body,
            tuple(jnp.int32(0) for _ in range(_NWIN + 1)))
        for k in range(_NWIN + 1):
            counts_s[k] = counts[k]

        def drain_scatter(b):
            pltpu.make_async_copy(
                rows_v.at[b],
                out_hbm.at[plsc.Indices(pos_v.at[b], ignored_value=-1)],
                semw,
            ).wait()

        def extract(k, nk, b, delta, climit, pend):
            @pl.when(pend > 0)
            def _():
                drain_scatter(b)

            def sb_body(sb, carry):
                @pl.when(sb > 0)
                def _():
                    drain_scatter(b)

                for q in range(_SB // _L):
                    pos_v[b, pl.ds(q * _L, _L)] = neg1

                def grp_body(g, carry2):
                    base = sb * _SB + g * _L
                    packed = wlist_v[pl.ds(k * _WPAD + base, _L)]
                    c16 = jnp.minimum(
                        lax.shift_right_logical(packed, 16) + delta, climit)
                    p16 = lax.bitwise_and(packed, 0x3FFF)
                    ok = (lane + base) < nk
                    pos_v[b, pl.ds(g * _L, _L)] = jnp.where(ok, p16, neg1)
                    slots = lane + g * _L
                    bvec = jnp.full((_L,), b, jnp.int32)
                    for col in range(_D):
                        ivec = jnp.full((_L,), col // 8, jnp.int32)
                        svec = jnp.full((_L,), col % 8, jnp.int32)
                        cvec = jnp.full((_L,), col, jnp.int32)
                        vals = plsc.load_gather(buf_v,
                                                [bvec, ivec, svec, c16])
                        plsc.store_scatter(rows_v, [bvec, slots, cvec],
                                           vals)
                    return carry2

                rem = jnp.minimum(nk - sb * _SB, _SB)
                lax.fori_loop(0, lax.div(rem + (_L - 1), _L), grp_body, 0)
                pltpu.async_copy(
                    rows_v.at[b],
                    out_hbm.at[plsc.Indices(pos_v.at[b], ignored_value=-1)],
                    semw,
                )
                return carry

            lax.fori_loop(0, lax.div(nk + (_SB - 1), _SB), sb_body, 0)
            return (nk > 0).astype(jnp.int32)

        def do_window(k, b, pend):
            drain(k, b)
            nk = counts_s[k]
            delta = (lo + k * _WIN) - win_start(k)
            newp = extract(k, nk, b, delta, _WIN - 1, pend)

            @pl.when(k + 2 < _NWIN)
            def _():
                issue(k + 2, b)

            return newp

        def pair_body(j, pends):
            p0 = do_window(2 * j, 0, pends[0])
            p1 = do_window(2 * j + 1, 1, pends[1])
            return (p0, p1)

        p0, p1 = lax.fori_loop(0, _NWIN // 2, pair_body,
                               (jnp.int32(0), jnp.int32(0)))
        p0 = do_window(_NWIN - 1, 0, p0)

        nt = counts_s[_NWIN]
        pltpu.sync_copy(tail_hbm, buf_v.at[1, :, :, pl.ds(0, 128)])
        p1 = extract(_NWIN, nt, 1, 0, 127, p1)

        @pl.when(p0 > 0)
        def _():
            drain_scatter(0)

        @pl.when(p1 > 0)
        def _():
            drain_scatter(1)

    return gather_kernel(table3, tail3, idx)


def _mlp_body(x_ref, w1_ref, b1_ref, w2_ref, b2_ref, o_ref):
    hT = lax.dot_general(w1_ref[...], x_ref[:, :_D],
                         (((0,), (1,)), ((), ())),
                         preferred_element_type=jnp.float32)
    hT = jnp.maximum(hT + b1_ref[...], 0.0)
    oT = lax.dot_general(w2_ref[...], hT, (((0,), (0,)), ((), ())),
                         preferred_element_type=jnp.float32)
    o_ref[...] = oT + b2_ref[...]


def _tc_mlp(x, W1, b1, W2, b2):
    return pl.pallas_call(
        _mlp_body,
        out_shape=jax.ShapeDtypeStruct((_D, _B), jnp.float32),
    )(x, W1, b1.reshape(_H, 1), W2, b2.reshape(_D, 1))


def kernel(channel_ids, table, W1, b1, W2, b2):
    tableT = table.T
    table3 = tableT.reshape(4, 8, _V)
    tail3 = jnp.pad(
        lax.slice(tableT, (0, _MAIN), (_D, _V)), ((0, 0), (0, 64))
    ).reshape(4, 8, 128)
    idx = channel_ids.astype(jnp.int32)
    rows = _sc_gather(table3, tail3, idx)
    return _tc_mlp(rows, W1, b1, W2, b2).T


# speedup vs baseline: 1.0117x; 1.0117x over previous
import functools

import jax
import jax.numpy as jnp
from jax import lax
from jax.experimental import pallas as pl
from jax.experimental.pallas import tpu as pltpu
from jax.experimental.pallas import tpu_sc as plsc

_B = 16384
_D = 32
_H = 64
_L = 16
_V = 1000000
_MAIN = 999936
_RANGE = 31360
_WIN = 768
_NWIN = 41
_WCAP = 256
_WPAD = 288
_SB = 32
_NCHUNK = 8
_CHK = 2048


def _sc_gather(table3, tail3, idx):
    mesh = plsc.VectorSubcoreMesh(core_axis_name="c", subcore_axis_name="s")

    @functools.partial(
        pl.kernel,
        mesh=mesh,
        out_type=jax.ShapeDtypeStruct((_B, 128), jnp.float32),
        scratch_types=[
            pltpu.VMEM((_B,), jnp.int32),
            pltpu.VMEM((_B + 32,), jnp.int32),
            pltpu.VMEM(((_NWIN + 1) * _WPAD,), jnp.int32),
            pltpu.VMEM((2, 4, 8, _WIN), jnp.float32),
            pltpu.VMEM((_SB, 128), jnp.float32),
            pltpu.VMEM((_SB,), jnp.int32),
            pltpu.SMEM((_NWIN + 1,), jnp.int32),
            pltpu.SemaphoreType.DMA,
            pltpu.SemaphoreType.DMA,
            pltpu.SemaphoreType.DMA,
        ],
        compiler_params=pltpu.CompilerParams(needs_layout_passes=False),
    )
    def gather_kernel(table_hbm, tail_hbm, idx_hbm, out_hbm, idx_v,
                      l1_v, wlist_v, buf_v, rows_v, pos_v, counts_s,
                      sem0, sem1, semw):
        wid = lax.axis_index("s") * 2 + lax.axis_index("c")
        lo = wid * _RANGE
        relmain = _MAIN - lo
        lane = lax.iota(jnp.int32, _L)
        neg1 = jnp.full((_L,), -1, jnp.int32)
        sems = (sem0, sem1)

        def win_start(k):
            return jnp.minimum(lo + k * _WIN, _MAIN - _WIN)

        def issue(k, b):
            pltpu.async_copy(
                table_hbm.at[:, :, pl.ds(win_start(k), _WIN)],
                buf_v.at[b], sems[b])

        def drain(k, b):
            pltpu.make_async_copy(
                table_hbm.at[:, :, pl.ds(win_start(k), _WIN)],
                buf_v.at[b], sems[b]).wait()

        issue(0, 0)
        issue(1, 1)

        pltpu.sync_copy(idx_hbm, idx_v)

        def l1_body(g, n):
            ids = idx_v[pl.ds(g * _L, _L)]
            rel = ids - lo
            mask = (rel >= 0) & (rel < _RANGE)
            pos = lane + g * _L
            packed = lax.shift_left(rel, 14) | pos
            plsc.store_compressed(l1_v.at[pl.ds(n, _L)], packed, mask=mask)
            return n + plsc.all_reduce_population_count(mask)[0]

        n1 = lax.fori_loop(0, _B // _L, l1_body, jnp.int32(0))

        def l2_body(g, counts):
            packed1 = l1_v[pl.ds(g * _L, _L)]
            rel = lax.shift_right_logical(packed1, 14)
            pos = lax.bitwise_and(packed1, 0x3FFF)
            valid = (lane + g * _L) < n1
            vmain = valid & (rel < relmain)
            out_counts = []
            for k in range(_NWIN):
                m = vmain & (rel >= k * _WIN) & (rel < (k + 1) * _WIN)
                packed = lax.shift_left(rel - k * _WIN, 16) | pos
                nk = counts[k]
                plsc.store_compressed(
                    wlist_v.at[pl.ds(k * _WPAD + nk, _L)], packed, mask=m)
                out_counts.append(
                    jnp.minimum(nk + plsc.all_reduce_population_count(m)[0],
                                _WCAP))
            m = valid & (rel >= relmain) & (rel < _RANGE)
            packed = lax.shift_left(rel - relmain, 16) | pos
            nk = counts[_NWIN]
            plsc.store_compressed(
                wlist_v.at[pl.ds(_NWIN * _WPAD + nk, _L)], packed, mask=m)
            out_counts.append(
                jnp.minimum(nk + plsc.all_reduce_population_count(m)[0],
                            _WCAP))
            return tuple(out_counts)

        counts = lax.fori_loop(
            0, lax.div(n1 + (_L - 1), _L), l2_body,
            tuple(jnp.int32(0) for _ in range(_NWIN + 1)))
        for k in range(_NWIN + 1):
            counts_s[k] = counts[k]

        def extract(k, nk, b, delta, climit):
            def sb_body(sb, carry):
                for q in range(_SB // _L):
                    pos_v[pl.ds(q * _L, _L)] = neg1

                def grp_body(g, carry2):
                    base = sb * _SB + g * _L
                    packed = wlist_v[pl.ds(k * _WPAD + base, _L)]
                    c16 = jnp.minimum(
                        lax.shift_right_logical(packed, 16) + delta, climit)
                    p16 = lax.bitwise_and(packed, 0x3FFF)
                    ok = (lane + base) < nk
                    pos_v[pl.ds(g * _L, _L)] = jnp.where(ok, p16, neg1)
                    slots = lane + g * _L
                    bvec = jnp.full((_L,), b, jnp.int32)
                    for col in range(_D):
                        ivec = jnp.full((_L,), col // 8, jnp.int32)
                        svec = jnp.full((_L,), col % 8, jnp.int32)
                        cvec = jnp.full((_L,), col, jnp.int32)
                        vals = plsc.load_gather(buf_v,
                                                [bvec, ivec, svec, c16])
                        plsc.store_scatter(rows_v, [slots, cvec], vals)
                    return carry2

                rem = jnp.minimum(nk - sb * _SB, _SB)
                lax.fori_loop(0, lax.div(rem + (_L - 1), _L), grp_body, 0)
                pltpu.async_copy(
                    rows_v,
                    out_hbm.at[plsc.Indices(pos_v, ignored_value=-1)],
                    semw,
                ).wait()
                return carry

            lax.fori_loop(0, lax.div(nk + (_SB - 1), _SB), sb_body, 0)

        def do_window(k, b):
            drain(k, b)
            nk = counts_s[k]
            delta = (lo + k * _WIN) - win_start(k)
            extract(k, nk, b, delta, _WIN - 1)

            @pl.when(k + 2 < _NWIN)
            def _():
                issue(k + 2, b)

        def pair_body(j, carry):
            do_window(2 * j, 0)
            do_window(2 * j + 1, 1)
            return carry

        lax.fori_loop(0, _NWIN // 2, pair_body, 0)
        do_window(_NWIN - 1, 0)

        nt = counts_s[_NWIN]

        @pl.when(nt > 0)
        def _():
            pltpu.sync_copy(tail_hbm, buf_v.at[1, :, :, pl.ds(0, 128)])
            extract(_NWIN, nt, 1, 0, 127)

    return gather_kernel(table3, tail3, idx)


def _mlp_body(x_ref, w1_ref, b1_ref, w2_ref, b2_ref, o_ref):
    hT = lax.dot_general(w1_ref[...], x_ref[:, :_D],
                         (((0,), (1,)), ((), ())),
                         preferred_element_type=jnp.float32)
    hT = jnp.maximum(hT + b1_ref[...], 0.0)
    oT = lax.dot_general(w2_ref[...], hT, (((0,), (0,)), ((), ())),
                         preferred_element_type=jnp.float32)
    o_ref[...] = oT + b2_ref[...]


def _tc_mlp(x, W1, b1, W2, b2):
    return pl.pallas_call(
        _mlp_body,
        out_shape=jax.ShapeDtypeStruct((_D, _B), jnp.float32),
    )(x, W1, b1.reshape(_H, 1), W2, b2.reshape(_D, 1))


def kernel(channel_ids, table, W1, b1, W2, b2):
    tableT = table.T
    table3 = tableT.reshape(4, 8, _V)
    tail3 = jnp.pad(
        lax.slice(tableT, (0, _MAIN), (_D, _V)), ((0, 0), (0, 64))
    ).reshape(4, 8, 128)
    idx = channel_ids.astype(jnp.int32)
    rows = _sc_gather(table3, tail3, idx)
    return _tc_mlp(rows, W1, b1, W2, b2).T
